# Initial kernel scaffold; baseline (speedup 1.0000x reference)
import jax
import jax.numpy as jnp
from jax.experimental import pallas as pl


def kernel(atom_types, bond_types, node_graph_indices, connectivity, atom_emb, bond_emb, abn_g, abn_b, abn_m, abn_v, bbn_g, bbn_b, bbn_m, bbn_v, bu1_W, bu2_W, bu2_b, au_W, o1_W, o1_b, o2_W, o2_b, last_W, last_b):
    raise NotImplementedError("write your pallas kernel here")


# speedup vs baseline: 3.9654x; 3.9654x over previous
import functools

import jax
import jax.numpy as jnp
from jax import lax
from jax.experimental import pallas as pl
from jax.experimental.pallas import tpu as pltpu
from jax.experimental.pallas import tpu_sc as plsc

N = 10000
E = 320000
D = 128
L = 3
G = 64

NC = 2
NS = 16
NW = NC * NS

TE = 2000
TN = 2000
EPW = E // NW
GED = 400
NG = EPW // GED
C = 80
CPG = GED // C

@functools.lru_cache(maxsize=None)
def _sc_mesh():
    return plsc.VectorSubcoreMesh(
        core_axis_name="c", subcore_axis_name="s", num_cores=NC, num_subcores=NS)


def _embed_body(types_ref, emb_ref, out_ref):
    t = types_ref[0]
    oh = (lax.broadcasted_iota(jnp.int32, (128, TN), 0) == t).astype(jnp.float32)
    out_ref[...] = lax.dot_general(
        oh, emb_ref[...], (((0,), (0,)), ((), ())),
        preferred_element_type=jnp.float32)


def _embed(types3, emb_pad, rows, tile):
    nblk = rows // tile
    return pl.pallas_call(
        _embed_body,
        grid=(nblk,),
        in_specs=[
            pl.BlockSpec((1, 1, tile), lambda i: (i, 0, 0)),
            pl.BlockSpec((128, 128), lambda i: (0, 0)),
        ],
        out_specs=pl.BlockSpec((tile, 128), lambda i: (i, 0)),
        out_shape=jax.ShapeDtypeStruct((rows, 128), jnp.float32),
    )(types3, emb_pad)


def _edge_math(s_raw, t_raw, b_raw, w1_ref, w2_ref, b2_ref, bnp_ref, auw_ref):
    asc = bnp_ref[0:1, :]
    ash = bnp_ref[1:2, :]
    bsc = bnp_ref[2:3, :]
    bsh = bnp_ref[3:4, :]
    s = s_raw * asc + ash
    t = t_raw * asc + ash
    b = b_raw * bsc + bsh
    z = (jnp.dot(s, w1_ref[0:128, :], preferred_element_type=jnp.float32)
         + jnp.dot(t, w1_ref[128:256, :], preferred_element_type=jnp.float32)
         + jnp.dot(b, w1_ref[256:384, :], preferred_element_type=jnp.float32))
    h = jax.nn.sigmoid(z)
    nb = jnp.dot(h, w2_ref[...], preferred_element_type=jnp.float32) + b2_ref[0:1, :]
    su = jax.nn.sigmoid(jnp.dot(s, auw_ref[...], preferred_element_type=jnp.float32))
    return b_raw + nb, su * nb


def _edge_body(src_ref, dst_ref, bond_ref, w1_ref, w2_ref, b2_ref, bnp_ref,
               auw_ref, bond_out_ref, msg_ref):
    bond_out_ref[...], msg_ref[...] = _edge_math(
        src_ref[...], dst_ref[...], bond_ref[...],
        w1_ref, w2_ref, b2_ref, bnp_ref, auw_ref)


def _edge_body0(src_ref, dst_ref, bt_ref, bemb_ref, w1_ref, w2_ref, b2_ref,
                bnp_ref, auw_ref, bond_out_ref, msg_ref):
    t = bt_ref[0]
    oh = (lax.broadcasted_iota(jnp.int32, (128, TE), 0) == t).astype(jnp.float32)
    b0 = lax.dot_general(oh, bemb_ref[...], (((0,), (0,)), ((), ())),
                         preferred_element_type=jnp.float32)
    bond_out_ref[...], msg_ref[...] = _edge_math(
        src_ref[...], dst_ref[...], b0,
        w1_ref, w2_ref, b2_ref, bnp_ref, auw_ref)


_W_SPECS = [
    pl.BlockSpec((384, 256), lambda i: (0, 0)),
    pl.BlockSpec((256, 128), lambda i: (0, 0)),
    pl.BlockSpec((1, 128), lambda i: (0, 0)),
    pl.BlockSpec((4, 128), lambda i: (0, 0)),
    pl.BlockSpec((128, 128), lambda i: (0, 0)),
]
_ROW_SPEC = pl.BlockSpec((TE, 128), lambda i: (i, 0))
_EDGE_OUT = dict(
    out_specs=[pl.BlockSpec((TE, 128), lambda i: (i, 0)),
               pl.BlockSpec((TE, 128), lambda i: (i, 0))],
    out_shape=[jax.ShapeDtypeStruct((E, 128), jnp.float32),
               jax.ShapeDtypeStruct((E, 128), jnp.float32)],
)


def _edge_layer(srows, drows, bond, w1, w2, b2, bnp, auw):
    return pl.pallas_call(
        _edge_body,
        grid=(E // TE,),
        in_specs=[_ROW_SPEC, _ROW_SPEC, _ROW_SPEC] + _W_SPECS,
        **_EDGE_OUT,
    )(srows, drows, bond, w1, w2, b2, bnp, auw)


def _edge_layer0(srows, drows, bt3, bemb_pad, w1, w2, b2, bnp, auw):
    return pl.pallas_call(
        _edge_body0,
        grid=(E // TE,),
        in_specs=[_ROW_SPEC, _ROW_SPEC,
                  pl.BlockSpec((1, 1, TE), lambda i: (i, 0, 0)),
                  pl.BlockSpec((128, 128), lambda i: (0, 0))] + _W_SPECS,
        **_EDGE_OUT,
    )(srows, drows, bt3, bemb_pad, w1, w2, b2, bnp, auw)


def _add3_body(a_ref, p0_ref, p1_ref, out_ref):
    out_ref[...] = a_ref[...] + p0_ref[...] + p1_ref[...]


def _add3(a, p0, p1):
    spec = pl.BlockSpec((TN, 128), lambda i: (i, 0))
    return pl.pallas_call(
        _add3_body,
        grid=(N // TN,),
        in_specs=[spec, spec, spec],
        out_specs=spec,
        out_shape=jax.ShapeDtypeStruct((N, 128), jnp.float32),
    )(a, p0, p1)


def _readout_body(a_ref, p0_ref, p1_ref, ngi_ref, o1w_ref, o1b_ref, o2w_ref,
                  o2b_ref, lw_ref, lb_ref, out_ref, acc_ref):
    j = pl.program_id(0)

    @pl.when(j == 0)
    def _():
        acc_ref[...] = jnp.zeros_like(acc_ref)

    a = a_ref[...] + p0_ref[...] + p1_ref[...]
    g = ngi_ref[0]
    oh = (lax.broadcasted_iota(jnp.int32, (G, TN), 0) == g).astype(jnp.float32)
    acc_ref[...] += jnp.dot(oh, a, preferred_element_type=jnp.float32)

    @pl.when(j == pl.num_programs(0) - 1)
    def _():
        m = jax.nn.relu(jnp.dot(acc_ref[...], o1w_ref[...],
                                preferred_element_type=jnp.float32) + o1b_ref[0:1, :])
        m = jax.nn.relu(jnp.dot(m, o2w_ref[...],
                                preferred_element_type=jnp.float32) + o2b_ref[0:1, :])
        out_ref[...] = jnp.dot(m, lw_ref[...],
                               preferred_element_type=jnp.float32) + lb_ref[0:1, :]


def _readout(a, p0, p1, ngi3, o1w, o1b, o2w, o2b, lw_pad, lb_pad):
    spec = pl.BlockSpec((TN, 128), lambda i: (i, 0))
    return pl.pallas_call(
        _readout_body,
        grid=(N // TN,),
        in_specs=[spec, spec, spec,
                  pl.BlockSpec((1, 1, TN), lambda i: (i, 0, 0)),
                  pl.BlockSpec((128, 128), lambda i: (0, 0)),
                  pl.BlockSpec((1, 128), lambda i: (0, 0)),
                  pl.BlockSpec((128, 64), lambda i: (0, 0)),
                  pl.BlockSpec((1, 64), lambda i: (0, 0)),
                  pl.BlockSpec((64, 128), lambda i: (0, 0)),
                  pl.BlockSpec((1, 128), lambda i: (0, 0))],
        out_specs=pl.BlockSpec((G, 128), lambda i: (0, 0)),
        out_shape=jax.ShapeDtypeStruct((G, 128), jnp.float32),
        scratch_shapes=[pltpu.VMEM((G, 128), jnp.float32)],
    )(a, p0, p1, ngi3, o1w, o1b, o2w, o2b, lw_pad, lb_pad)


def _gather_body(table, src_hbm, dst_hbm, srows_out, drows_out,
                 idx_s, idx_d, rows_s, rows_d, sem):
    wid = lax.axis_index("s") * NC + lax.axis_index("c")
    wbase = wid * EPW

    def group(g, _):
        base = wbase + g * GED
        pltpu.sync_copy(src_hbm.at[pl.ds(base, GED)], idx_s)
        pltpu.sync_copy(dst_hbm.at[pl.ds(base, GED)], idx_d)
        descs = []
        for j in range(CPG):
            sl = pl.ds(j * C, C)
            descs.append(pltpu.async_copy(
                table.at[idx_s.at[sl]], rows_s.at[sl, :], sem))
            descs.append(pltpu.async_copy(
                table.at[idx_d.at[sl]], rows_d.at[sl, :], sem))
        for dsc in descs:
            dsc.wait()
        pltpu.sync_copy(rows_s, srows_out.at[pl.ds(base, GED), :])
        pltpu.sync_copy(rows_d, drows_out.at[pl.ds(base, GED), :])
        return ()

    lax.fori_loop(0, NG, group, (), unroll=False)


@functools.lru_cache(maxsize=None)
def _gather2():
    return pl.kernel(
        _gather_body,
        out_type=(jax.ShapeDtypeStruct((E, 128), jnp.float32),
                  jax.ShapeDtypeStruct((E, 128), jnp.float32)),
        mesh=_sc_mesh(),
        scratch_types=[
            pltpu.VMEM((GED,), jnp.int32),
            pltpu.VMEM((GED,), jnp.int32),
            pltpu.VMEM((GED, 128), jnp.float32),
            pltpu.VMEM((GED, 128), jnp.float32),
            pltpu.SemaphoreType.DMA,
        ],
    )


NSTR = E // 128
SPW = NSTR // NW
NBUF = 2


def _scatter_body(msg_hbm, dst_hbm, zeros_hbm, p0_out, p1_out,
                  acc, rows, idx0, idx1, sem_m0, sem_m1, sem_i0, sem_i1):
    cid = lax.axis_index("c")
    sid = lax.axis_index("s")
    wid = sid * NC + cid

    @pl.when(sid == 0)
    def _():
        pltpu.sync_copy(zeros_hbm, acc)

    plsc.subcore_barrier()

    sems_m = (sem_m0, sem_m1)
    sems_i = (sem_i0, sem_i1)
    idxs = (idx0, idx1)
    s0 = wid * SPW

    def load(s, b):
        pltpu.async_copy(msg_hbm.at[pl.ds(s * 128, 128), :],
                         rows.at[b], sems_m[b])
        pltpu.async_copy(dst_hbm.at[pl.ds(s * 128, 128)], idxs[b], sems_i[b])

    def wait(s, b):
        pltpu.make_async_copy(msg_hbm.at[pl.ds(s * 128, 128), :],
                              rows.at[b], sems_m[b]).wait()
        pltpu.make_async_copy(dst_hbm.at[pl.ds(s * 128, 128)], idxs[b],
                              sems_i[b]).wait()

    def add(b):
        pltpu.sync_copy(rows.at[b], acc.at[idxs[b]], add=True)

    for b in range(NBUF):
        load(s0 + b, b)

    def group(g, _):
        for b in range(NBUF):
            it = g * NBUF + b
            s = s0 + it
            wait(s, b)
            add(b)

            @pl.when(it + NBUF < SPW)
            def _():
                load(s + NBUF, b)
        return ()

    lax.fori_loop(0, SPW // NBUF, group, (), unroll=False)

    @pl.when(wid < NSTR - NW * SPW)
    def _():
        s = NW * SPW + wid
        load(s, 0)
        wait(s, 0)
        add(0)

    plsc.subcore_barrier()

    slab = 624
    sl = pl.ds(sid * slab, slab)

    @pl.when(cid == 0)
    def _():
        pltpu.sync_copy(acc.at[sl, :], p0_out.at[sl, :])

        @pl.when(sid == NS - 1)
        def _():
            tl = pl.ds((NS - 1) * slab + slab, N - NS * slab)
            pltpu.sync_copy(acc.at[tl, :], p0_out.at[tl, :])

    @pl.when(cid == 1)
    def _():
        pltpu.sync_copy(acc.at[sl, :], p1_out.at[sl, :])

        @pl.when(sid == NS - 1)
        def _():
            tl = pl.ds((NS - 1) * slab + slab, N - NS * slab)
            pltpu.sync_copy(acc.at[tl, :], p1_out.at[tl, :])


@functools.lru_cache(maxsize=None)
def _scatter():
    return pl.kernel(
        _scatter_body,
        out_type=(jax.ShapeDtypeStruct((N, 128), jnp.float32),
                  jax.ShapeDtypeStruct((N, 128), jnp.float32)),
        mesh=_sc_mesh(),
        scratch_types=[
            pltpu.VMEM_SHARED((N, 128), jnp.float32),
            pltpu.VMEM((NBUF, 128, 128), jnp.float32),
            pltpu.VMEM((128,), jnp.int32),
            pltpu.VMEM((128,), jnp.int32),
            pltpu.SemaphoreType.DMA,
            pltpu.SemaphoreType.DMA,
            pltpu.SemaphoreType.DMA,
            pltpu.SemaphoreType.DMA,
        ],
    )


def kernel(atom_types, bond_types, node_graph_indices, connectivity,
           atom_emb, bond_emb,
           abn_g, abn_b, abn_m, abn_v,
           bbn_g, bbn_b, bbn_m, bbn_v,
           bu1_W, bu2_W, bu2_b, au_W,
           o1_W, o1_b, o2_W, o2_b, last_W, last_b):
    f32 = jnp.float32
    dst = connectivity[:, 0].astype(jnp.int32)
    src = connectivity[:, 1].astype(jnp.int32)
    at3 = atom_types.astype(jnp.int32).reshape(N // TN, 1, TN)
    bt3 = bond_types.astype(jnp.int32).reshape(E // TE, 1, TE)
    ngi3 = node_graph_indices.astype(jnp.int32).reshape(N // TN, 1, TN)

    aemb_pad = jnp.zeros((128, 128), f32).at[:atom_emb.shape[0]].set(atom_emb)
    bemb_pad = jnp.zeros((128, 128), f32).at[:bond_emb.shape[0]].set(bond_emb)
    lw_pad = jnp.zeros((64, 128), f32).at[:, :1].set(last_W)
    lb_pad = jnp.zeros((128,), f32).at[:1].set(last_b).reshape(1, 128)
    zeros_n = jnp.zeros((N, 128), f32)

    a_sc = abn_g / jnp.sqrt(abn_v + 1e-3)
    a_sh = abn_b - abn_m * a_sc
    b_sc = bbn_g / jnp.sqrt(bbn_v + 1e-3)
    b_sh = bbn_b - bbn_m * b_sc
    bnp = jnp.stack([a_sc, a_sh, b_sc, b_sh], axis=1)

    atom_state = _embed(at3, aemb_pad, N, TN)

    bond_state = None
    for i in range(L):
        srows, drows = _gather2()(atom_state, src, dst)
        w1 = bu1_W[i]
        w2 = bu2_W[i]
        b2 = bu2_b[i].reshape(1, D)
        auw = au_W[i]
        if i == 0:
            bond_state, msg = _edge_layer0(
                srows, drows, bt3, bemb_pad, w1, w2, b2, bnp[i], auw)
        else:
            bond_state, msg = _edge_layer(
                srows, drows, bond_state, w1, w2, b2, bnp[i], auw)
        p0, p1 = _scatter()(msg, dst, zeros_n)
        if i < L - 1:
            atom_state = _add3(atom_state, p0, p1)

    out = _readout(atom_state, p0, p1, ngi3,
                   o1_W, o1_b.reshape(1, 128), o2_W, o2_b.reshape(1, 64),
                   lw_pad, lb_pad)
    return out[:, :1]


# speedup vs baseline: 3.9773x; 1.0030x over previous
import functools

import jax
import jax.numpy as jnp
from jax import lax
from jax.experimental import pallas as pl
from jax.experimental.pallas import tpu as pltpu
from jax.experimental.pallas import tpu_sc as plsc

N = 10000
E = 320000
D = 128
L = 3
G = 64

NC = 2
NS = 16
NW = NC * NS

TE = 2000
TN = 2000
EPW = E // NW
GED = 400
NG = EPW // GED
C = 80
CPG = GED // C

@functools.lru_cache(maxsize=None)
def _sc_mesh():
    return plsc.VectorSubcoreMesh(
        core_axis_name="c", subcore_axis_name="s", num_cores=NC, num_subcores=NS)


def _embed_body(types_ref, emb_ref, out_ref):
    t = types_ref[0]
    oh = (lax.broadcasted_iota(jnp.int32, (128, TN), 0) == t).astype(jnp.float32)
    out_ref[...] = lax.dot_general(
        oh, emb_ref[...], (((0,), (0,)), ((), ())),
        preferred_element_type=jnp.float32)


def _embed(types3, emb_pad, rows, tile):
    nblk = rows // tile
    return pl.pallas_call(
        _embed_body,
        grid=(nblk,),
        in_specs=[
            pl.BlockSpec((1, 1, tile), lambda i: (i, 0, 0)),
            pl.BlockSpec((128, 128), lambda i: (0, 0)),
        ],
        out_specs=pl.BlockSpec((tile, 128), lambda i: (i, 0)),
        out_shape=jax.ShapeDtypeStruct((rows, 128), jnp.float32),
    )(types3, emb_pad)


def _edge_math(s16, t16, b_f32, w1_ref, w2_ref, b2_ref, zc_ref, auw_ref,
               suc_ref):
    s = s16.astype(jnp.float32)
    t = t16.astype(jnp.float32)
    z = (jnp.dot(s, w1_ref[0:128, :], preferred_element_type=jnp.float32)
         + jnp.dot(t, w1_ref[128:256, :], preferred_element_type=jnp.float32)
         + jnp.dot(b_f32, w1_ref[256:384, :], preferred_element_type=jnp.float32)
         + zc_ref[0:1, :])
    h = jax.nn.sigmoid(z)
    nb = jnp.dot(h, w2_ref[...], preferred_element_type=jnp.float32) + b2_ref[0:1, :]
    su = jax.nn.sigmoid(
        jnp.dot(s, auw_ref[...], preferred_element_type=jnp.float32)
        + suc_ref[0:1, :])
    return b_f32 + nb, su * nb


def _edge_body(src_ref, dst_ref, bond_ref, w1_ref, w2_ref, b2_ref, zc_ref,
               auw_ref, suc_ref, bond_out_ref, msg_ref):
    bond_out_ref[...], msg_ref[...] = _edge_math(
        src_ref[...], dst_ref[...], bond_ref[...],
        w1_ref, w2_ref, b2_ref, zc_ref, auw_ref, suc_ref)


def _edge_body0(src_ref, dst_ref, bt_ref, bemb_ref, w1_ref, w2_ref, b2_ref,
                zc_ref, auw_ref, suc_ref, bond_out_ref, msg_ref):
    t = bt_ref[0]
    oh = (lax.broadcasted_iota(jnp.int32, (128, TE), 0) == t).astype(jnp.float32)
    b0 = lax.dot_general(oh, bemb_ref[...], (((0,), (0,)), ((), ())),
                         preferred_element_type=jnp.float32)
    bond_out_ref[...], msg_ref[...] = _edge_math(
        src_ref[...], dst_ref[...], b0,
        w1_ref, w2_ref, b2_ref, zc_ref, auw_ref, suc_ref)


_W_SPECS = [
    pl.BlockSpec((384, 256), lambda i: (0, 0)),
    pl.BlockSpec((256, 128), lambda i: (0, 0)),
    pl.BlockSpec((1, 128), lambda i: (0, 0)),
    pl.BlockSpec((1, 256), lambda i: (0, 0)),
    pl.BlockSpec((128, 128), lambda i: (0, 0)),
    pl.BlockSpec((1, 128), lambda i: (0, 0)),
]
_ROW_SPEC = pl.BlockSpec((TE, 128), lambda i: (i, 0))
_EDGE_OUT = dict(
    out_specs=[pl.BlockSpec((TE, 128), lambda i: (i, 0)),
               pl.BlockSpec((TE, 128), lambda i: (i, 0))],
    out_shape=[jax.ShapeDtypeStruct((E, 128), jnp.float32),
               jax.ShapeDtypeStruct((E, 128), jnp.float32)],
)


def _edge_layer(srows, drows, bond, wargs):
    return pl.pallas_call(
        _edge_body,
        grid=(E // TE,),
        in_specs=[_ROW_SPEC, _ROW_SPEC, _ROW_SPEC] + _W_SPECS,
        **_EDGE_OUT,
    )(srows, drows, bond, *wargs)


def _edge_layer0(srows, drows, bt3, bemb_pad, wargs):
    return pl.pallas_call(
        _edge_body0,
        grid=(E // TE,),
        in_specs=[_ROW_SPEC, _ROW_SPEC,
                  pl.BlockSpec((1, 1, TE), lambda i: (i, 0, 0)),
                  pl.BlockSpec((128, 128), lambda i: (0, 0))] + _W_SPECS,
        **_EDGE_OUT,
    )(srows, drows, bt3, bemb_pad, *wargs)


def _add3_body(a_ref, p0_ref, p1_ref, out_ref):
    out_ref[...] = a_ref[...] + p0_ref[...] + p1_ref[...]


def _add3(a, p0, p1):
    spec = pl.BlockSpec((TN, 128), lambda i: (i, 0))
    return pl.pallas_call(
        _add3_body,
        grid=(N // TN,),
        in_specs=[spec, spec, spec],
        out_specs=spec,
        out_shape=jax.ShapeDtypeStruct((N, 128), jnp.float32),
    )(a, p0, p1)


def _readout_body(a_ref, p0_ref, p1_ref, ngi_ref, o1w_ref, o1b_ref, o2w_ref,
                  o2b_ref, lw_ref, lb_ref, out_ref, acc_ref):
    j = pl.program_id(0)

    @pl.when(j == 0)
    def _():
        acc_ref[...] = jnp.zeros_like(acc_ref)

    a = a_ref[...] + p0_ref[...] + p1_ref[...]
    g = ngi_ref[0]
    oh = (lax.broadcasted_iota(jnp.int32, (G, TN), 0) == g).astype(jnp.float32)
    acc_ref[...] += jnp.dot(oh, a, preferred_element_type=jnp.float32)

    @pl.when(j == pl.num_programs(0) - 1)
    def _():
        m = jax.nn.relu(jnp.dot(acc_ref[...], o1w_ref[...],
                                preferred_element_type=jnp.float32) + o1b_ref[0:1, :])
        m = jax.nn.relu(jnp.dot(m, o2w_ref[...],
                                preferred_element_type=jnp.float32) + o2b_ref[0:1, :])
        out_ref[...] = jnp.dot(m, lw_ref[...],
                               preferred_element_type=jnp.float32) + lb_ref[0:1, :]


def _readout(a, p0, p1, ngi3, o1w, o1b, o2w, o2b, lw_pad, lb_pad):
    spec = pl.BlockSpec((TN, 128), lambda i: (i, 0))
    return pl.pallas_call(
        _readout_body,
        grid=(N // TN,),
        in_specs=[spec, spec, spec,
                  pl.BlockSpec((1, 1, TN), lambda i: (i, 0, 0)),
                  pl.BlockSpec((128, 128), lambda i: (0, 0)),
                  pl.BlockSpec((1, 128), lambda i: (0, 0)),
                  pl.BlockSpec((128, 64), lambda i: (0, 0)),
                  pl.BlockSpec((1, 64), lambda i: (0, 0)),
                  pl.BlockSpec((64, 128), lambda i: (0, 0)),
                  pl.BlockSpec((1, 128), lambda i: (0, 0))],
        out_specs=pl.BlockSpec((G, 128), lambda i: (0, 0)),
        out_shape=jax.ShapeDtypeStruct((G, 128), jnp.float32),
        scratch_shapes=[pltpu.VMEM((G, 128), jnp.float32)],
    )(a, p0, p1, ngi3, o1w, o1b, o2w, o2b, lw_pad, lb_pad)


def _gather_body(table, src_hbm, dst_hbm, srows_out, drows_out,
                 idx_s, idx_d, rows_s, rows_d, sem):
    wid = lax.axis_index("s") * NC + lax.axis_index("c")
    wbase = wid * EPW

    def group(g, _):
        base = wbase + g * GED
        pltpu.sync_copy(src_hbm.at[pl.ds(base, GED)], idx_s)
        pltpu.sync_copy(dst_hbm.at[pl.ds(base, GED)], idx_d)
        descs = []
        for j in range(CPG):
            sl = pl.ds(j * C, C)
            descs.append(pltpu.async_copy(
                table.at[idx_s.at[sl]], rows_s.at[sl, :], sem))
            descs.append(pltpu.async_copy(
                table.at[idx_d.at[sl]], rows_d.at[sl, :], sem))
        for dsc in descs:
            dsc.wait()
        pltpu.sync_copy(rows_s, srows_out.at[pl.ds(base, GED), :])
        pltpu.sync_copy(rows_d, drows_out.at[pl.ds(base, GED), :])
        return ()

    lax.fori_loop(0, NG, group, (), unroll=False)


@functools.lru_cache(maxsize=None)
def _gather2():
    return pl.kernel(
        _gather_body,
        out_type=(jax.ShapeDtypeStruct((E, 128), jnp.float32),
                  jax.ShapeDtypeStruct((E, 128), jnp.float32)),
        mesh=_sc_mesh(),
        scratch_types=[
            pltpu.VMEM((GED,), jnp.int32),
            pltpu.VMEM((GED,), jnp.int32),
            pltpu.VMEM((GED, 128), jnp.float32),
            pltpu.VMEM((GED, 128), jnp.float32),
            pltpu.SemaphoreType.DMA,
        ],
    )


NSTR = E // 128
SPW = NSTR // NW
NBUF = 2


def _scatter_body(msg_hbm, dst_hbm, zeros_hbm, p0_out, p1_out,
                  acc, rows, idx0, idx1, sem_m0, sem_m1, sem_i0, sem_i1):
    cid = lax.axis_index("c")
    sid = lax.axis_index("s")
    wid = sid * NC + cid

    @pl.when(sid == 0)
    def _():
        pltpu.sync_copy(zeros_hbm, acc)

    plsc.subcore_barrier()

    sems_m = (sem_m0, sem_m1)
    sems_i = (sem_i0, sem_i1)
    idxs = (idx0, idx1)
    s0 = wid * SPW

    def load(s, b):
        pltpu.async_copy(msg_hbm.at[pl.ds(s * 128, 128), :],
                         rows.at[b], sems_m[b])
        pltpu.async_copy(dst_hbm.at[pl.ds(s * 128, 128)], idxs[b], sems_i[b])

    def wait(s, b):
        pltpu.make_async_copy(msg_hbm.at[pl.ds(s * 128, 128), :],
                              rows.at[b], sems_m[b]).wait()
        pltpu.make_async_copy(dst_hbm.at[pl.ds(s * 128, 128)], idxs[b],
                              sems_i[b]).wait()

    def add(b):
        pltpu.sync_copy(rows.at[b], acc.at[idxs[b]], add=True)

    for b in range(NBUF):
        load(s0 + b, b)

    def group(g, _):
        for b in range(NBUF):
            it = g * NBUF + b
            s = s0 + it
            wait(s, b)
            add(b)

            @pl.when(it + NBUF < SPW)
            def _():
                load(s + NBUF, b)
        return ()

    lax.fori_loop(0, SPW // NBUF, group, (), unroll=False)

    @pl.when(wid < NSTR - NW * SPW)
    def _():
        s = NW * SPW + wid
        load(s, 0)
        wait(s, 0)
        add(0)

    plsc.subcore_barrier()

    slab = 624
    sl = pl.ds(sid * slab, slab)

    @pl.when(cid == 0)
    def _():
        pltpu.sync_copy(acc.at[sl, :], p0_out.at[sl, :])

        @pl.when(sid == NS - 1)
        def _():
            tl = pl.ds((NS - 1) * slab + slab, N - NS * slab)
            pltpu.sync_copy(acc.at[tl, :], p0_out.at[tl, :])

    @pl.when(cid == 1)
    def _():
        pltpu.sync_copy(acc.at[sl, :], p1_out.at[sl, :])

        @pl.when(sid == NS - 1)
        def _():
            tl = pl.ds((NS - 1) * slab + slab, N - NS * slab)
            pltpu.sync_copy(acc.at[tl, :], p1_out.at[tl, :])


@functools.lru_cache(maxsize=None)
def _scatter():
    return pl.kernel(
        _scatter_body,
        out_type=(jax.ShapeDtypeStruct((N, 128), jnp.float32),
                  jax.ShapeDtypeStruct((N, 128), jnp.float32)),
        mesh=_sc_mesh(),
        scratch_types=[
            pltpu.VMEM_SHARED((N, 128), jnp.float32),
            pltpu.VMEM((NBUF, 128, 128), jnp.float32),
            pltpu.VMEM((128,), jnp.int32),
            pltpu.VMEM((128,), jnp.int32),
            pltpu.SemaphoreType.DMA,
            pltpu.SemaphoreType.DMA,
            pltpu.SemaphoreType.DMA,
            pltpu.SemaphoreType.DMA,
        ],
    )


def kernel(atom_types, bond_types, node_graph_indices, connectivity,
           atom_emb, bond_emb,
           abn_g, abn_b, abn_m, abn_v,
           bbn_g, bbn_b, bbn_m, bbn_v,
           bu1_W, bu2_W, bu2_b, au_W,
           o1_W, o1_b, o2_W, o2_b, last_W, last_b):
    f32 = jnp.float32
    dst = connectivity[:, 0].astype(jnp.int32)
    src = connectivity[:, 1].astype(jnp.int32)
    at3 = atom_types.astype(jnp.int32).reshape(N // TN, 1, TN)
    bt3 = bond_types.astype(jnp.int32).reshape(E // TE, 1, TE)
    ngi3 = node_graph_indices.astype(jnp.int32).reshape(N // TN, 1, TN)

    aemb_pad = jnp.zeros((128, 128), f32).at[:atom_emb.shape[0]].set(atom_emb)
    bemb_pad = jnp.zeros((128, 128), f32).at[:bond_emb.shape[0]].set(bond_emb)
    lw_pad = jnp.zeros((64, 128), f32).at[:, :1].set(last_W)
    lb_pad = jnp.zeros((128,), f32).at[:1].set(last_b).reshape(1, 128)
    zeros_n = jnp.zeros((N, 128), f32)

    a_sc = abn_g / jnp.sqrt(abn_v + 1e-3)
    a_sh = abn_b - abn_m * a_sc
    b_sc = bbn_g / jnp.sqrt(bbn_v + 1e-3)
    b_sh = bbn_b - bbn_m * b_sc

    atom_state = _embed(at3, aemb_pad, N, TN)

    bond_state = None
    for i in range(L):
        srows, drows = _gather2()(atom_state, src, dst)
        w1 = bu1_W[i]
        w1p = jnp.concatenate([
            w1[0:128] * a_sc[i][:, None],
            w1[128:256] * a_sc[i][:, None],
            w1[256:384] * b_sc[i][:, None],
        ])
        zc = (a_sh[i] @ (w1[0:128] + w1[128:256])
              + b_sh[i] @ w1[256:384]).reshape(1, 2 * D)
        aup = au_W[i] * a_sc[i][:, None]
        suc = (a_sh[i] @ au_W[i]).reshape(1, D)
        wargs = (w1p, bu2_W[i], bu2_b[i].reshape(1, D), zc, aup, suc)
        if i == 0:
            bond_state, msg = _edge_layer0(srows, drows, bt3, bemb_pad, wargs)
        else:
            bond_state, msg = _edge_layer(srows, drows, bond_state, wargs)
        p0, p1 = _scatter()(msg, dst, zeros_n)
        if i < L - 1:
            atom_state = _add3(atom_state, p0, p1)

    out = _readout(atom_state, p0, p1, ngi3,
                   o1_W, o1_b.reshape(1, 128), o2_W, o2_b.reshape(1, 64),
                   lw_pad, lb_pad)
    return out[:, :1]


# speedup vs baseline: 4.4576x; 1.1208x over previous
import functools

import jax
import jax.numpy as jnp
from jax import lax
from jax.experimental import pallas as pl
from jax.experimental.pallas import tpu as pltpu
from jax.experimental.pallas import tpu_sc as plsc

N = 10000
E = 320000
D = 128
L = 3
G = 64

NC = 2
NS = 16
NW = NC * NS

NCHUNK = 2
EC = E // NCHUNK

TE = 2000
TN = 2000

EPW = EC // NW
GED = 200
NG = EPW // GED
C = 40
CPG = GED // C

SW = 128
NSTR = EC // SW
SPW = NSTR // NW
SLEFT = NSTR - NW * SPW


@functools.lru_cache(maxsize=None)
def _sc_mesh():
    return plsc.VectorSubcoreMesh(
        core_axis_name="c", subcore_axis_name="s", num_cores=NC, num_subcores=NS)


def _embed_body(types_ref, emb_ref, out_ref):
    t = types_ref[0]
    oh = (lax.broadcasted_iota(jnp.int32, (128, TN), 0) == t).astype(jnp.float32)
    out_ref[...] = lax.dot_general(
        oh, emb_ref[...], (((0,), (0,)), ((), ())),
        preferred_element_type=jnp.float32)


def _embed(types3, emb_pad, rows, tile):
    nblk = rows // tile
    return pl.pallas_call(
        _embed_body,
        grid=(nblk,),
        in_specs=[
            pl.BlockSpec((1, 1, tile), lambda i: (i, 0, 0)),
            pl.BlockSpec((128, 128), lambda i: (0, 0)),
        ],
        out_specs=pl.BlockSpec((tile, 128), lambda i: (i, 0)),
        out_shape=jax.ShapeDtypeStruct((rows, 128), jnp.float32),
    )(types3, emb_pad)


def _edge_math(s, t, b_f32, w1_ref, w2_ref, b2_ref, zc_ref, auw_ref, suc_ref):
    z = (jnp.dot(s, w1_ref[0:128, :], preferred_element_type=jnp.float32)
         + jnp.dot(t, w1_ref[128:256, :], preferred_element_type=jnp.float32)
         + jnp.dot(b_f32, w1_ref[256:384, :], preferred_element_type=jnp.float32)
         + zc_ref[0:1, :])
    h = jax.nn.sigmoid(z)
    nb = jnp.dot(h, w2_ref[...], preferred_element_type=jnp.float32) + b2_ref[0:1, :]
    su = jax.nn.sigmoid(
        jnp.dot(s, auw_ref[...], preferred_element_type=jnp.float32)
        + suc_ref[0:1, :])
    return b_f32 + nb, su * nb


def _edge_body(src_ref, dst_ref, bond_ref, w1_ref, w2_ref, b2_ref, zc_ref,
               auw_ref, suc_ref, bond_out_ref, msg_ref):
    bond_out_ref[...], msg_ref[...] = _edge_math(
        src_ref[...], dst_ref[...], bond_ref[...],
        w1_ref, w2_ref, b2_ref, zc_ref, auw_ref, suc_ref)


def _edge_body0(src_ref, dst_ref, bt_ref, bemb_ref, w1_ref, w2_ref, b2_ref,
                zc_ref, auw_ref, suc_ref, bond_out_ref, msg_ref):
    t = bt_ref[0]
    oh = (lax.broadcasted_iota(jnp.int32, (128, TE), 0) == t).astype(jnp.float32)
    b0 = lax.dot_general(oh, bemb_ref[...], (((0,), (0,)), ((), ())),
                         preferred_element_type=jnp.float32)
    bond_out_ref[...], msg_ref[...] = _edge_math(
        src_ref[...], dst_ref[...], b0,
        w1_ref, w2_ref, b2_ref, zc_ref, auw_ref, suc_ref)


_W_SPECS = [
    pl.BlockSpec((384, 256), lambda i: (0, 0)),
    pl.BlockSpec((256, 128), lambda i: (0, 0)),
    pl.BlockSpec((1, 128), lambda i: (0, 0)),
    pl.BlockSpec((1, 256), lambda i: (0, 0)),
    pl.BlockSpec((128, 128), lambda i: (0, 0)),
    pl.BlockSpec((1, 128), lambda i: (0, 0)),
]
_ROW_SPEC = pl.BlockSpec((TE, 128), lambda i: (i, 0))
_EDGE_OUT = dict(
    out_specs=[pl.BlockSpec((TE, 128), lambda i: (i, 0)),
               pl.BlockSpec((TE, 128), lambda i: (i, 0))],
    out_shape=[jax.ShapeDtypeStruct((EC, 128), jnp.float32),
               jax.ShapeDtypeStruct((EC, 128), jnp.float32)],
)


def _edge_layer(srows, drows, bond, wargs):
    return pl.pallas_call(
        _edge_body,
        grid=(EC // TE,),
        in_specs=[_ROW_SPEC, _ROW_SPEC, _ROW_SPEC] + _W_SPECS,
        **_EDGE_OUT,
    )(srows, drows, bond, *wargs)


def _edge_layer0(srows, drows, bt3, bemb_pad, wargs):
    return pl.pallas_call(
        _edge_body0,
        grid=(EC // TE,),
        in_specs=[_ROW_SPEC, _ROW_SPEC,
                  pl.BlockSpec((1, 1, TE), lambda i: (i, 0, 0)),
                  pl.BlockSpec((128, 128), lambda i: (0, 0))] + _W_SPECS,
        **_EDGE_OUT,
    )(srows, drows, bt3, bemb_pad, *wargs)


def _add5_body(a_ref, p0_ref, p1_ref, p2_ref, p3_ref, out_ref):
    out_ref[...] = (a_ref[...] + ((p0_ref[...] + p1_ref[...])
                                  + (p2_ref[...] + p3_ref[...])))


def _add5(a, ps):
    spec = pl.BlockSpec((TN, 128), lambda i: (i, 0))
    return pl.pallas_call(
        _add5_body,
        grid=(N // TN,),
        in_specs=[spec] * 5,
        out_specs=spec,
        out_shape=jax.ShapeDtypeStruct((N, 128), jnp.float32),
    )(a, *ps)


def _readout_body(a_ref, p0_ref, p1_ref, p2_ref, p3_ref, ngi_ref, o1w_ref,
                  o1b_ref, o2w_ref, o2b_ref, lw_ref, lb_ref, out_ref, acc_ref):
    j = pl.program_id(0)

    @pl.when(j == 0)
    def _():
        acc_ref[...] = jnp.zeros_like(acc_ref)

    a = a_ref[...] + ((p0_ref[...] + p1_ref[...]) + (p2_ref[...] + p3_ref[...]))
    g = ngi_ref[0]
    oh = (lax.broadcasted_iota(jnp.int32, (G, TN), 0) == g).astype(jnp.float32)
    acc_ref[...] += jnp.dot(oh, a, preferred_element_type=jnp.float32)

    @pl.when(j == pl.num_programs(0) - 1)
    def _():
        m = jax.nn.relu(jnp.dot(acc_ref[...], o1w_ref[...],
                                preferred_element_type=jnp.float32) + o1b_ref[0:1, :])
        m = jax.nn.relu(jnp.dot(m, o2w_ref[...],
                                preferred_element_type=jnp.float32) + o2b_ref[0:1, :])
        out_ref[...] = jnp.dot(m, lw_ref[...],
                               preferred_element_type=jnp.float32) + lb_ref[0:1, :]


def _readout(a, ps, ngi3, o1w, o1b, o2w, o2b, lw_pad, lb_pad):
    spec = pl.BlockSpec((TN, 128), lambda i: (i, 0))
    return pl.pallas_call(
        _readout_body,
        grid=(N // TN,),
        in_specs=[spec, spec, spec, spec, spec,
                  pl.BlockSpec((1, 1, TN), lambda i: (i, 0, 0)),
                  pl.BlockSpec((128, 128), lambda i: (0, 0)),
                  pl.BlockSpec((1, 128), lambda i: (0, 0)),
                  pl.BlockSpec((128, 64), lambda i: (0, 0)),
                  pl.BlockSpec((1, 64), lambda i: (0, 0)),
                  pl.BlockSpec((64, 128), lambda i: (0, 0)),
                  pl.BlockSpec((1, 128), lambda i: (0, 0))],
        out_specs=pl.BlockSpec((G, 128), lambda i: (0, 0)),
        out_shape=jax.ShapeDtypeStruct((G, 128), jnp.float32),
        scratch_shapes=[pltpu.VMEM((G, 128), jnp.float32)],
    )(a, *ps, ngi3, o1w, o1b, o2w, o2b, lw_pad, lb_pad)


def _make_gather(e0):
    def _gather_body(table, src_hbm, dst_hbm, srows_out, drows_out,
                     idx_s, idx_d, rows_s, rows_d,
                     sem_g, sem_os0, sem_os1, sem_od0, sem_od1):
        wid = lax.axis_index("s") * NC + lax.axis_index("c")
        wloc = wid * EPW
        pltpu.sync_copy(src_hbm.at[pl.ds(e0 + wloc, EPW)], idx_s)
        pltpu.sync_copy(dst_hbm.at[pl.ds(e0 + wloc, EPW)], idx_d)
        sem_os = (sem_os0, sem_os1)
        sem_od = (sem_od0, sem_od1)

        def out_start(g, b):
            gb = pl.ds(wloc + g * GED, GED)
            pltpu.async_copy(rows_s.at[b], srows_out.at[gb, :], sem_os[b])
            pltpu.async_copy(rows_d.at[b], drows_out.at[gb, :], sem_od[b])

        def out_wait(g, b):
            gb = pl.ds(wloc + g * GED, GED)
            pltpu.make_async_copy(rows_s.at[b], srows_out.at[gb, :],
                                  sem_os[b]).wait()
            pltpu.make_async_copy(rows_d.at[b], drows_out.at[gb, :],
                                  sem_od[b]).wait()

        def do_group(g, b):
            descs = []
            for j in range(CPG):
                sl = pl.ds(g * GED + j * C, C)
                bl = pl.ds(j * C, C)
                descs.append(pltpu.async_copy(
                    table.at[idx_s.at[sl]], rows_s.at[b, bl, :], sem_g))
                descs.append(pltpu.async_copy(
                    table.at[idx_d.at[sl]], rows_d.at[b, bl, :], sem_g))
            for dsc in descs:
                dsc.wait()
            out_start(g, b)

        def pair(gg, _):
            for b in range(2):
                g = gg * 2 + b

                @pl.when(gg > 0)
                def _():
                    out_wait(g - 2, b)

                do_group(g, b)
            return ()

        lax.fori_loop(0, NG // 2, pair, (), unroll=False)
        out_wait(NG - 3, 0)
        do_group(NG - 1, 0)
        out_wait(NG - 2, 1)
        out_wait(NG - 1, 0)

    return pl.kernel(
        _gather_body,
        out_type=(jax.ShapeDtypeStruct((EC, 128), jnp.float32),
                  jax.ShapeDtypeStruct((EC, 128), jnp.float32)),
        mesh=_sc_mesh(),
        scratch_types=[
            pltpu.VMEM((EPW,), jnp.int32),
            pltpu.VMEM((EPW,), jnp.int32),
            pltpu.VMEM((2, GED, 128), jnp.float32),
            pltpu.VMEM((2, GED, 128), jnp.float32),
            pltpu.SemaphoreType.DMA,
            pltpu.SemaphoreType.DMA,
            pltpu.SemaphoreType.DMA,
            pltpu.SemaphoreType.DMA,
            pltpu.SemaphoreType.DMA,
        ],
    )


@functools.lru_cache(maxsize=None)
def _gather_chunk(e0):
    return _make_gather(e0)


def _gather_call(e0, table, src, dst):
    return _gather_chunk(e0)(table, src, dst)


NBUF = 2


def _make_scatter():
    def _scatter_body(msg_hbm, dst_hbm, zeros_hbm, p0_out, p1_out,
                      acc, rows, idx0, idx1, sem_m0, sem_m1, sem_i0, sem_i1):
        cid = lax.axis_index("c")
        sid = lax.axis_index("s")
        wid = sid * NC + cid

        @pl.when(sid == 0)
        def _():
            pltpu.sync_copy(zeros_hbm, acc)

        plsc.subcore_barrier()

        sems_m = (sem_m0, sem_m1)
        sems_i = (sem_i0, sem_i1)
        idxs = (idx0, idx1)
        s0 = wid * SPW

        def load(s, b):
            pltpu.async_copy(msg_hbm.at[pl.ds(s * SW, SW), :],
                             rows.at[b], sems_m[b])
            pltpu.async_copy(dst_hbm.at[pl.ds(s * SW, SW)], idxs[b], sems_i[b])

        def wait(s, b):
            pltpu.make_async_copy(msg_hbm.at[pl.ds(s * SW, SW), :],
                                  rows.at[b], sems_m[b]).wait()
            pltpu.make_async_copy(dst_hbm.at[pl.ds(s * SW, SW)], idxs[b],
                                  sems_i[b]).wait()

        def add(b):
            pltpu.sync_copy(rows.at[b], acc.at[idxs[b]], add=True)

        for b in range(NBUF):
            load(s0 + b, b)

        def group(g, _):
            for b in range(NBUF):
                it = g * NBUF + b
                s = s0 + it
                wait(s, b)
                add(b)

                @pl.when(it + NBUF < SPW)
                def _():
                    load(s + NBUF, b)
            return ()

        lax.fori_loop(0, SPW // NBUF, group, (), unroll=False)

        s = s0 + SPW - 1
        wait(s, (SPW - 1) % NBUF)
        add((SPW - 1) % NBUF)

        @pl.when(wid < SLEFT)
        def _():
            s = NW * SPW + wid
            load(s, 0)
            wait(s, 0)
            add(0)

        plsc.subcore_barrier()

        slab = 624
        sl = pl.ds(sid * slab, slab)

        @pl.when(cid == 0)
        def _():
            pltpu.sync_copy(acc.at[sl, :], p0_out.at[sl, :])

            @pl.when(sid == NS - 1)
            def _():
                tl = pl.ds(NS * slab, N - NS * slab)
                pltpu.sync_copy(acc.at[tl, :], p0_out.at[tl, :])

        @pl.when(cid == 1)
        def _():
            pltpu.sync_copy(acc.at[sl, :], p1_out.at[sl, :])

            @pl.when(sid == NS - 1)
            def _():
                tl = pl.ds(NS * slab, N - NS * slab)
                pltpu.sync_copy(acc.at[tl, :], p1_out.at[tl, :])

    return pl.kernel(
        _scatter_body,
        out_type=(jax.ShapeDtypeStruct((N, 128), jnp.float32),
                  jax.ShapeDtypeStruct((N, 128), jnp.float32)),
        mesh=_sc_mesh(),
        scratch_types=[
            pltpu.VMEM_SHARED((N, 128), jnp.float32),
            pltpu.VMEM((NBUF, SW, 128), jnp.float32),
            pltpu.VMEM((SW,), jnp.int32),
            pltpu.VMEM((SW,), jnp.int32),
            pltpu.SemaphoreType.DMA,
            pltpu.SemaphoreType.DMA,
            pltpu.SemaphoreType.DMA,
            pltpu.SemaphoreType.DMA,
        ],
    )


@functools.lru_cache(maxsize=None)
def _scatter_kernel():
    return _make_scatter()


def _scatter_call(msg_c, dst_c, zeros_n):
    return _scatter_kernel()(msg_c, dst_c, zeros_n)


def kernel(atom_types, bond_types, node_graph_indices, connectivity,
           atom_emb, bond_emb,
           abn_g, abn_b, abn_m, abn_v,
           bbn_g, bbn_b, bbn_m, bbn_v,
           bu1_W, bu2_W, bu2_b, au_W,
           o1_W, o1_b, o2_W, o2_b, last_W, last_b):
    f32 = jnp.float32
    dst = connectivity[:, 0].astype(jnp.int32)
    src = connectivity[:, 1].astype(jnp.int32)
    dst_c = [lax.slice_in_dim(dst, c * EC, (c + 1) * EC) for c in range(NCHUNK)]
    at3 = atom_types.astype(jnp.int32).reshape(N // TN, 1, TN)
    bt3 = bond_types.astype(jnp.int32).reshape(E // TE, 1, TE)
    bt3_c = [bt3[c * (EC // TE):(c + 1) * (EC // TE)] for c in range(NCHUNK)]
    ngi3 = node_graph_indices.astype(jnp.int32).reshape(N // TN, 1, TN)

    aemb_pad = jnp.zeros((128, 128), f32).at[:atom_emb.shape[0]].set(atom_emb)
    bemb_pad = jnp.zeros((128, 128), f32).at[:bond_emb.shape[0]].set(bond_emb)
    lw_pad = jnp.zeros((64, 128), f32).at[:, :1].set(last_W)
    lb_pad = jnp.zeros((128,), f32).at[:1].set(last_b).reshape(1, 128)
    zeros_n = jnp.zeros((N, 128), f32)

    a_sc = abn_g / jnp.sqrt(abn_v + 1e-3)
    a_sh = abn_b - abn_m * a_sc
    b_sc = bbn_g / jnp.sqrt(bbn_v + 1e-3)
    b_sh = bbn_b - bbn_m * b_sc

    atom_state = _embed(at3, aemb_pad, N, TN)

    bond_c = [None] * NCHUNK
    for i in range(L):
        w1 = bu1_W[i]
        w1p = jnp.concatenate([
            w1[0:128] * a_sc[i][:, None],
            w1[128:256] * a_sc[i][:, None],
            w1[256:384] * b_sc[i][:, None],
        ])
        zc = (a_sh[i] @ (w1[0:128] + w1[128:256])
              + b_sh[i] @ w1[256:384]).reshape(1, 2 * D)
        aup = au_W[i] * a_sc[i][:, None]
        suc = (a_sh[i] @ au_W[i]).reshape(1, D)
        wargs = (w1p, bu2_W[i], bu2_b[i].reshape(1, D), zc, aup, suc)

        rows_c = [_gather_call(c * EC, atom_state, src, dst)
                  for c in range(NCHUNK)]
        msg_c = [None] * NCHUNK
        for c in range(NCHUNK):
            srows, drows = rows_c[c]
            if i == 0:
                bond_c[c], msg_c[c] = _edge_layer0(
                    srows, drows, bt3_c[c], bemb_pad, wargs)
            else:
                bond_c[c], msg_c[c] = _edge_layer(
                    srows, drows, bond_c[c], wargs)
        parts = []
        for c in range(NCHUNK):
            parts.extend(_scatter_call(msg_c[c], dst_c[c], zeros_n))
        if i < L - 1:
            atom_state = _add5(atom_state, parts)

    out = _readout(atom_state, parts, ngi3,
                   o1_W, o1_b.reshape(1, 128), o2_W, o2_b.reshape(1, 64),
                   lw_pad, lb_pad)
    return out[:, :1]


# speedup vs baseline: 4.4987x; 1.0092x over previous
import functools

import jax
import jax.numpy as jnp
from jax import lax
from jax.experimental import pallas as pl
from jax.experimental.pallas import tpu as pltpu
from jax.experimental.pallas import tpu_sc as plsc

N = 10000
E = 320000
D = 128
L = 3
G = 64

NC = 2
NS = 16
NW = NC * NS

NCHUNK = 2
EC = E // NCHUNK

TE = 2000
TN = 2000

EPW = EC // NW
GED = 200
NG = EPW // GED
C = 40
CPG = GED // C

SW = 128
NSTR = EC // SW
SPW = NSTR // NW
SLEFT = NSTR - NW * SPW


@functools.lru_cache(maxsize=None)
def _sc_mesh():
    return plsc.VectorSubcoreMesh(
        core_axis_name="c", subcore_axis_name="s", num_cores=NC, num_subcores=NS)


def _embed_body(types_ref, emb_ref, out_ref):
    t = types_ref[0]
    oh = (lax.broadcasted_iota(jnp.int32, (128, TN), 0) == t).astype(jnp.float32)
    out_ref[...] = lax.dot_general(
        oh, emb_ref[...], (((0,), (0,)), ((), ())),
        preferred_element_type=jnp.float32)


def _embed(types3, emb_pad, rows, tile):
    nblk = rows // tile
    return pl.pallas_call(
        _embed_body,
        grid=(nblk,),
        in_specs=[
            pl.BlockSpec((1, 1, tile), lambda i: (i, 0, 0)),
            pl.BlockSpec((128, 128), lambda i: (0, 0)),
        ],
        out_specs=pl.BlockSpec((tile, 128), lambda i: (i, 0)),
        out_shape=jax.ShapeDtypeStruct((rows, 128), jnp.float32),
    )(types3, emb_pad)


def _edge_math(s, t, b_f32, w1_ref, w2_ref, b2_ref, zc_ref, auw_ref, suc_ref):
    bf = jnp.bfloat16
    s16 = s.astype(bf)
    t16 = t.astype(bf)
    b16 = b_f32.astype(bf)
    z = (jnp.dot(s16, w1_ref[0:128, :], preferred_element_type=jnp.float32)
         + jnp.dot(t16, w1_ref[128:256, :], preferred_element_type=jnp.float32)
         + jnp.dot(b16, w1_ref[256:384, :], preferred_element_type=jnp.float32)
         + zc_ref[0:1, :])
    h = jax.nn.sigmoid(z)
    nb = jnp.dot(h, w2_ref[...], preferred_element_type=jnp.float32) + b2_ref[0:1, :]
    su = jax.nn.sigmoid(
        jnp.dot(s16, auw_ref[...], preferred_element_type=jnp.float32)
        + suc_ref[0:1, :])
    return (b_f32 + nb).astype(jnp.bfloat16), su * nb


def _edge_body(src_ref, dst_ref, bond_ref, w1_ref, w2_ref, b2_ref, zc_ref,
               auw_ref, suc_ref, bond_out_ref, msg_ref):
    bond_out_ref[...], msg_ref[...] = _edge_math(
        src_ref[...], dst_ref[...], bond_ref[...].astype(jnp.float32),
        w1_ref, w2_ref, b2_ref, zc_ref, auw_ref, suc_ref)


def _edge_body0(src_ref, dst_ref, bt_ref, bemb_ref, w1_ref, w2_ref, b2_ref,
                zc_ref, auw_ref, suc_ref, bond_out_ref, msg_ref):
    t = bt_ref[0]
    oh = (lax.broadcasted_iota(jnp.int32, (128, TE), 0) == t).astype(jnp.float32)
    b0 = lax.dot_general(oh, bemb_ref[...], (((0,), (0,)), ((), ())),
                         preferred_element_type=jnp.float32)
    bond_out_ref[...], msg_ref[...] = _edge_math(
        src_ref[...], dst_ref[...], b0,
        w1_ref, w2_ref, b2_ref, zc_ref, auw_ref, suc_ref)


_W_SPECS = [
    pl.BlockSpec((384, 256), lambda i: (0, 0)),
    pl.BlockSpec((256, 128), lambda i: (0, 0)),
    pl.BlockSpec((1, 128), lambda i: (0, 0)),
    pl.BlockSpec((1, 256), lambda i: (0, 0)),
    pl.BlockSpec((128, 128), lambda i: (0, 0)),
    pl.BlockSpec((1, 128), lambda i: (0, 0)),
]
_ROW_SPEC = pl.BlockSpec((TE, 128), lambda i: (i, 0))
_EDGE_OUT = dict(
    out_specs=[pl.BlockSpec((TE, 128), lambda i: (i, 0)),
               pl.BlockSpec((TE, 128), lambda i: (i, 0))],
    out_shape=[jax.ShapeDtypeStruct((EC, 128), jnp.bfloat16),
               jax.ShapeDtypeStruct((EC, 128), jnp.float32)],
)


def _edge_layer(srows, drows, bond, wargs):
    return pl.pallas_call(
        _edge_body,
        grid=(EC // TE,),
        in_specs=[_ROW_SPEC, _ROW_SPEC, _ROW_SPEC] + _W_SPECS,
        **_EDGE_OUT,
    )(srows, drows, bond, *wargs)


def _edge_layer0(srows, drows, bt3, bemb_pad, wargs):
    return pl.pallas_call(
        _edge_body0,
        grid=(EC // TE,),
        in_specs=[_ROW_SPEC, _ROW_SPEC,
                  pl.BlockSpec((1, 1, TE), lambda i: (i, 0, 0)),
                  pl.BlockSpec((128, 128), lambda i: (0, 0))] + _W_SPECS,
        **_EDGE_OUT,
    )(srows, drows, bt3, bemb_pad, *wargs)


def _add5_body(a_ref, p0_ref, p1_ref, p2_ref, p3_ref, out_ref):
    out_ref[...] = (a_ref[...] + ((p0_ref[...] + p1_ref[...])
                                  + (p2_ref[...] + p3_ref[...])))


def _add5(a, ps):
    spec = pl.BlockSpec((TN, 128), lambda i: (i, 0))
    return pl.pallas_call(
        _add5_body,
        grid=(N // TN,),
        in_specs=[spec] * 5,
        out_specs=spec,
        out_shape=jax.ShapeDtypeStruct((N, 128), jnp.float32),
    )(a, *ps)


def _readout_body(a_ref, p0_ref, p1_ref, p2_ref, p3_ref, ngi_ref, o1w_ref,
                  o1b_ref, o2w_ref, o2b_ref, lw_ref, lb_ref, out_ref, acc_ref):
    j = pl.program_id(0)

    @pl.when(j == 0)
    def _():
        acc_ref[...] = jnp.zeros_like(acc_ref)

    a = a_ref[...] + ((p0_ref[...] + p1_ref[...]) + (p2_ref[...] + p3_ref[...]))
    g = ngi_ref[0]
    oh = (lax.broadcasted_iota(jnp.int32, (G, TN), 0) == g).astype(jnp.float32)
    acc_ref[...] += jnp.dot(oh, a, preferred_element_type=jnp.float32)

    @pl.when(j == pl.num_programs(0) - 1)
    def _():
        m = jax.nn.relu(jnp.dot(acc_ref[...], o1w_ref[...],
                                preferred_element_type=jnp.float32) + o1b_ref[0:1, :])
        m = jax.nn.relu(jnp.dot(m, o2w_ref[...],
                                preferred_element_type=jnp.float32) + o2b_ref[0:1, :])
        out_ref[...] = jnp.dot(m, lw_ref[...],
                               preferred_element_type=jnp.float32) + lb_ref[0:1, :]


def _readout(a, ps, ngi3, o1w, o1b, o2w, o2b, lw_pad, lb_pad):
    spec = pl.BlockSpec((TN, 128), lambda i: (i, 0))
    return pl.pallas_call(
        _readout_body,
        grid=(N // TN,),
        in_specs=[spec, spec, spec, spec, spec,
                  pl.BlockSpec((1, 1, TN), lambda i: (i, 0, 0)),
                  pl.BlockSpec((128, 128), lambda i: (0, 0)),
                  pl.BlockSpec((1, 128), lambda i: (0, 0)),
                  pl.BlockSpec((128, 64), lambda i: (0, 0)),
                  pl.BlockSpec((1, 64), lambda i: (0, 0)),
                  pl.BlockSpec((64, 128), lambda i: (0, 0)),
                  pl.BlockSpec((1, 128), lambda i: (0, 0))],
        out_specs=pl.BlockSpec((G, 128), lambda i: (0, 0)),
        out_shape=jax.ShapeDtypeStruct((G, 128), jnp.float32),
        scratch_shapes=[pltpu.VMEM((G, 128), jnp.float32)],
    )(a, *ps, ngi3, o1w, o1b, o2w, o2b, lw_pad, lb_pad)


def _make_gather(e0):
    def _gather_body(table, src_hbm, dst_hbm, srows_out, drows_out,
                     idx_s, idx_d, rows_s, rows_d,
                     sem_g0, sem_g1, sem_os0, sem_os1, sem_od0, sem_od1):
        wid = lax.axis_index("s") * NC + lax.axis_index("c")
        wloc = wid * EPW
        pltpu.sync_copy(src_hbm.at[pl.ds(e0 + wloc, EPW)], idx_s)
        pltpu.sync_copy(dst_hbm.at[pl.ds(e0 + wloc, EPW)], idx_d)
        sem_g = (sem_g0, sem_g1)
        sem_os = (sem_os0, sem_os1)
        sem_od = (sem_od0, sem_od1)

        def out_start(g, b):
            gb = pl.ds(wloc + g * GED, GED)
            pltpu.async_copy(rows_s.at[b], srows_out.at[gb, :], sem_os[b])
            pltpu.async_copy(rows_d.at[b], drows_out.at[gb, :], sem_od[b])

        def out_wait(g, b):
            gb = pl.ds(wloc + g * GED, GED)
            pltpu.make_async_copy(rows_s.at[b], srows_out.at[gb, :],
                                  sem_os[b]).wait()
            pltpu.make_async_copy(rows_d.at[b], drows_out.at[gb, :],
                                  sem_od[b]).wait()

        def g_start(g, b):
            for j in range(CPG):
                sl = pl.ds(g * GED + j * C, C)
                bl = pl.ds(j * C, C)
                pltpu.async_copy(table.at[idx_s.at[sl]], rows_s.at[b, bl, :],
                                 sem_g[b])
                pltpu.async_copy(table.at[idx_d.at[sl]], rows_d.at[b, bl, :],
                                 sem_g[b])

        def g_wait(g, b):
            for j in range(CPG):
                sl = pl.ds(g * GED + j * C, C)
                bl = pl.ds(j * C, C)
                pltpu.make_async_copy(table.at[idx_s.at[sl]],
                                      rows_s.at[b, bl, :], sem_g[b]).wait()
                pltpu.make_async_copy(table.at[idx_d.at[sl]],
                                      rows_d.at[b, bl, :], sem_g[b]).wait()

        g_start(0, 0)

        def pair(gg, _):
            for b in range(2):
                g = gg * 2 + b

                @pl.when(g < NG)
                def _():
                    @pl.when(g + 1 < NG)
                    def _():
                        @pl.when(g >= 1)
                        def _():
                            out_wait(g - 1, 1 - b)

                        g_start(g + 1, 1 - b)

                    g_wait(g, b)
                    out_start(g, b)
            return ()

        lax.fori_loop(0, (NG + 1) // 2, pair, (), unroll=False)
        out_wait(NG - 2, (NG - 2) % 2)
        out_wait(NG - 1, (NG - 1) % 2)

    return pl.kernel(
        _gather_body,
        out_type=(jax.ShapeDtypeStruct((EC, 128), jnp.float32),
                  jax.ShapeDtypeStruct((EC, 128), jnp.float32)),
        mesh=_sc_mesh(),
        scratch_types=[
            pltpu.VMEM((EPW,), jnp.int32),
            pltpu.VMEM((EPW,), jnp.int32),
            pltpu.VMEM((2, GED, 128), jnp.float32),
            pltpu.VMEM((2, GED, 128), jnp.float32),
            pltpu.SemaphoreType.DMA,
            pltpu.SemaphoreType.DMA,
            pltpu.SemaphoreType.DMA,
            pltpu.SemaphoreType.DMA,
            pltpu.SemaphoreType.DMA,
            pltpu.SemaphoreType.DMA,
        ],
    )


@functools.lru_cache(maxsize=None)
def _gather_chunk(e0):
    return _make_gather(e0)


def _gather_call(e0, table, src, dst):
    return _gather_chunk(e0)(table, src, dst)


NBUF = 2


def _make_scatter():
    def _scatter_body(msg_hbm, dst_hbm, zeros_hbm, p0_out, p1_out,
                      acc, rows, idx0, idx1, sem_m0, sem_m1, sem_i0, sem_i1):
        cid = lax.axis_index("c")
        sid = lax.axis_index("s")
        wid = sid * NC + cid

        @pl.when(sid == 0)
        def _():
            pltpu.sync_copy(zeros_hbm, acc)

        plsc.subcore_barrier()

        sems_m = (sem_m0, sem_m1)
        sems_i = (sem_i0, sem_i1)
        idxs = (idx0, idx1)
        s0 = wid * SPW

        def load(s, b):
            pltpu.async_copy(msg_hbm.at[pl.ds(s * SW, SW), :],
                             rows.at[b], sems_m[b])
            pltpu.async_copy(dst_hbm.at[pl.ds(s * SW, SW)], idxs[b], sems_i[b])

        def wait(s, b):
            pltpu.make_async_copy(msg_hbm.at[pl.ds(s * SW, SW), :],
                                  rows.at[b], sems_m[b]).wait()
            pltpu.make_async_copy(dst_hbm.at[pl.ds(s * SW, SW)], idxs[b],
                                  sems_i[b]).wait()

        def add(b):
            pltpu.sync_copy(rows.at[b], acc.at[idxs[b]], add=True)

        for b in range(NBUF):
            load(s0 + b, b)

        def group(g, _):
            for b in range(NBUF):
                it = g * NBUF + b
                s = s0 + it
                wait(s, b)
                add(b)

                @pl.when(it + NBUF < SPW)
                def _():
                    load(s + NBUF, b)
            return ()

        lax.fori_loop(0, SPW // NBUF, group, (), unroll=False)

        s = s0 + SPW - 1
        wait(s, (SPW - 1) % NBUF)
        add((SPW - 1) % NBUF)

        @pl.when(wid < SLEFT)
        def _():
            s = NW * SPW + wid
            load(s, 0)
            wait(s, 0)
            add(0)

        plsc.subcore_barrier()

        slab = 624
        sl = pl.ds(sid * slab, slab)

        @pl.when(cid == 0)
        def _():
            pltpu.sync_copy(acc.at[sl, :], p0_out.at[sl, :])

            @pl.when(sid == NS - 1)
            def _():
                tl = pl.ds(NS * slab, N - NS * slab)
                pltpu.sync_copy(acc.at[tl, :], p0_out.at[tl, :])

        @pl.when(cid == 1)
        def _():
            pltpu.sync_copy(acc.at[sl, :], p1_out.at[sl, :])

            @pl.when(sid == NS - 1)
            def _():
                tl = pl.ds(NS * slab, N - NS * slab)
                pltpu.sync_copy(acc.at[tl, :], p1_out.at[tl, :])

    return pl.kernel(
        _scatter_body,
        out_type=(jax.ShapeDtypeStruct((N, 128), jnp.float32),
                  jax.ShapeDtypeStruct((N, 128), jnp.float32)),
        mesh=_sc_mesh(),
        scratch_types=[
            pltpu.VMEM_SHARED((N, 128), jnp.float32),
            pltpu.VMEM((NBUF, SW, 128), jnp.float32),
            pltpu.VMEM((SW,), jnp.int32),
            pltpu.VMEM((SW,), jnp.int32),
            pltpu.SemaphoreType.DMA,
            pltpu.SemaphoreType.DMA,
            pltpu.SemaphoreType.DMA,
            pltpu.SemaphoreType.DMA,
        ],
    )


@functools.lru_cache(maxsize=None)
def _scatter_kernel():
    return _make_scatter()


def _scatter_call(msg_c, dst_c, zeros_n):
    return _scatter_kernel()(msg_c, dst_c, zeros_n)


def kernel(atom_types, bond_types, node_graph_indices, connectivity,
           atom_emb, bond_emb,
           abn_g, abn_b, abn_m, abn_v,
           bbn_g, bbn_b, bbn_m, bbn_v,
           bu1_W, bu2_W, bu2_b, au_W,
           o1_W, o1_b, o2_W, o2_b, last_W, last_b):
    f32 = jnp.float32
    dst = connectivity[:, 0].astype(jnp.int32)
    src = connectivity[:, 1].astype(jnp.int32)
    dst_c = [lax.slice_in_dim(dst, c * EC, (c + 1) * EC) for c in range(NCHUNK)]
    at3 = atom_types.astype(jnp.int32).reshape(N // TN, 1, TN)
    bt3 = bond_types.astype(jnp.int32).reshape(E // TE, 1, TE)
    bt3_c = [bt3[c * (EC // TE):(c + 1) * (EC // TE)] for c in range(NCHUNK)]
    ngi3 = node_graph_indices.astype(jnp.int32).reshape(N // TN, 1, TN)

    aemb_pad = jnp.zeros((128, 128), f32).at[:atom_emb.shape[0]].set(atom_emb)
    bemb_pad = jnp.zeros((128, 128), f32).at[:bond_emb.shape[0]].set(bond_emb)
    lw_pad = jnp.zeros((64, 128), f32).at[:, :1].set(last_W)
    lb_pad = jnp.zeros((128,), f32).at[:1].set(last_b).reshape(1, 128)
    zeros_n = jnp.zeros((N, 128), f32)

    a_sc = abn_g / jnp.sqrt(abn_v + 1e-3)
    a_sh = abn_b - abn_m * a_sc
    b_sc = bbn_g / jnp.sqrt(bbn_v + 1e-3)
    b_sh = bbn_b - bbn_m * b_sc

    atom_state = _embed(at3, aemb_pad, N, TN)

    bond_c = [None] * NCHUNK
    for i in range(L):
        w1 = bu1_W[i]
        w1p = jnp.concatenate([
            w1[0:128] * a_sc[i][:, None],
            w1[128:256] * a_sc[i][:, None],
            w1[256:384] * b_sc[i][:, None],
        ]).astype(jnp.bfloat16)
        zc = (a_sh[i] @ (w1[0:128] + w1[128:256])
              + b_sh[i] @ w1[256:384]).reshape(1, 2 * D)
        aup = (au_W[i] * a_sc[i][:, None]).astype(jnp.bfloat16)
        suc = (a_sh[i] @ au_W[i]).reshape(1, D)
        wargs = (w1p, bu2_W[i], bu2_b[i].reshape(1, D), zc, aup, suc)

        rows_c = [_gather_call(c * EC, atom_state, src, dst)
                  for c in range(NCHUNK)]
        msg_c = [None] * NCHUNK
        for c in range(NCHUNK):
            srows, drows = rows_c[c]
            if i == 0:
                bond_c[c], msg_c[c] = _edge_layer0(
                    srows, drows, bt3_c[c], bemb_pad, wargs)
            else:
                bond_c[c], msg_c[c] = _edge_layer(
                    srows, drows, bond_c[c], wargs)
        parts = []
        for c in range(NCHUNK):
            parts.extend(_scatter_call(msg_c[c], dst_c[c], zeros_n))
        if i < L - 1:
            atom_state = _add5(atom_state, parts)

    out = _readout(atom_state, parts, ngi3,
                   o1_W, o1_b.reshape(1, 128), o2_W, o2_b.reshape(1, 64),
                   lw_pad, lb_pad)
    return out[:, :1]


# speedup vs baseline: 4.9819x; 1.1074x over previous
import functools

import jax
import jax.numpy as jnp
from jax import lax
from jax.experimental import pallas as pl
from jax.experimental.pallas import tpu as pltpu
from jax.experimental.pallas import tpu_sc as plsc

N = 10000
E = 320000
D = 128
L = 3
G = 64

NC = 2
NS = 16
NW = NC * NS

NCHUNK = 2
EC = E // NCHUNK

TE = 4000
TN = 2000

EPW = EC // NW
GED = 200
NG = EPW // GED
C = 40
CPG = GED // C

SW = 128
NSTR = EC // SW
SPW = NSTR // NW
SLEFT = NSTR - NW * SPW


@functools.lru_cache(maxsize=None)
def _sc_mesh():
    return plsc.VectorSubcoreMesh(
        core_axis_name="c", subcore_axis_name="s", num_cores=NC, num_subcores=NS)


def _embed_body(types_ref, emb_ref, out_ref):
    t = types_ref[0]
    oh = (lax.broadcasted_iota(jnp.int32, (128, TN), 0) == t).astype(jnp.float32)
    out_ref[...] = lax.dot_general(
        oh, emb_ref[...], (((0,), (0,)), ((), ())),
        preferred_element_type=jnp.float32)


def _embed(types3, emb_pad, rows, tile):
    nblk = rows // tile
    return pl.pallas_call(
        _embed_body,
        grid=(nblk,),
        in_specs=[
            pl.BlockSpec((1, 1, tile), lambda i: (i, 0, 0)),
            pl.BlockSpec((128, 128), lambda i: (0, 0)),
        ],
        out_specs=pl.BlockSpec((tile, 128), lambda i: (i, 0)),
        out_shape=jax.ShapeDtypeStruct((rows, 128), jnp.float32),
    )(types3, emb_pad)


def _edge_math(s, t, b_f32, w1_ref, w2_ref, b2_ref, zc_ref, auw_ref, suc_ref):
    bf = jnp.bfloat16
    s16 = s.astype(bf)
    t16 = t.astype(bf)
    b16 = b_f32.astype(bf)
    z = (jnp.dot(s16, w1_ref[0:128, :], preferred_element_type=jnp.float32)
         + jnp.dot(t16, w1_ref[128:256, :], preferred_element_type=jnp.float32)
         + jnp.dot(b16, w1_ref[256:384, :], preferred_element_type=jnp.float32)
         + zc_ref[0:1, :])
    h = jax.nn.sigmoid(z)
    nb = jnp.dot(h, w2_ref[...], preferred_element_type=jnp.float32) + b2_ref[0:1, :]
    su = jax.nn.sigmoid(
        jnp.dot(s16, auw_ref[...], preferred_element_type=jnp.float32)
        + suc_ref[0:1, :])
    return (b_f32 + nb).astype(jnp.bfloat16), su * nb


def _edge_body(src_ref, dst_ref, bond_ref, w1_ref, w2_ref, b2_ref, zc_ref,
               auw_ref, suc_ref, bond_out_ref, msg_ref):
    bond_out_ref[...], msg_ref[...] = _edge_math(
        src_ref[...], dst_ref[...], bond_ref[...].astype(jnp.float32),
        w1_ref, w2_ref, b2_ref, zc_ref, auw_ref, suc_ref)


def _edge_body0(src_ref, dst_ref, bt_ref, bemb_ref, w1_ref, w2_ref, b2_ref,
                zc_ref, auw_ref, suc_ref, bond_out_ref, msg_ref):
    t = bt_ref[0]
    oh = (lax.broadcasted_iota(jnp.int32, (128, TE), 0) == t).astype(jnp.float32)
    b0 = lax.dot_general(oh, bemb_ref[...], (((0,), (0,)), ((), ())),
                         preferred_element_type=jnp.float32)
    bond_out_ref[...], msg_ref[...] = _edge_math(
        src_ref[...], dst_ref[...], b0,
        w1_ref, w2_ref, b2_ref, zc_ref, auw_ref, suc_ref)


_W_SPECS = [
    pl.BlockSpec((384, 256), lambda i: (0, 0)),
    pl.BlockSpec((256, 128), lambda i: (0, 0)),
    pl.BlockSpec((1, 128), lambda i: (0, 0)),
    pl.BlockSpec((1, 256), lambda i: (0, 0)),
    pl.BlockSpec((128, 128), lambda i: (0, 0)),
    pl.BlockSpec((1, 128), lambda i: (0, 0)),
]
_ROW_SPEC = pl.BlockSpec((TE, 128), lambda i: (i, 0))
_EDGE_OUT = dict(
    out_specs=[pl.BlockSpec((TE, 128), lambda i: (i, 0)),
               pl.BlockSpec((TE, 128), lambda i: (i, 0))],
    out_shape=[jax.ShapeDtypeStruct((EC, 128), jnp.bfloat16),
               jax.ShapeDtypeStruct((EC, 128), jnp.float32)],
)


def _edge_layer(srows, drows, bond, wargs):
    return pl.pallas_call(
        _edge_body,
        grid=(EC // TE,),
        in_specs=[_ROW_SPEC, _ROW_SPEC, _ROW_SPEC] + _W_SPECS,
        **_EDGE_OUT,
    )(srows, drows, bond, *wargs)


def _edge_layer0(srows, drows, bt3, bemb_pad, wargs):
    return pl.pallas_call(
        _edge_body0,
        grid=(EC // TE,),
        in_specs=[_ROW_SPEC, _ROW_SPEC,
                  pl.BlockSpec((1, 1, TE), lambda i: (i, 0, 0)),
                  pl.BlockSpec((128, 128), lambda i: (0, 0))] + _W_SPECS,
        **_EDGE_OUT,
    )(srows, drows, bt3, bemb_pad, *wargs)


def _add5_body(a_ref, p0_ref, p1_ref, p2_ref, p3_ref, out_ref):
    out_ref[...] = (a_ref[...] + ((p0_ref[...] + p1_ref[...])
                                  + (p2_ref[...] + p3_ref[...])))


def _add5(a, ps):
    spec = pl.BlockSpec((TN, 128), lambda i: (i, 0))
    return pl.pallas_call(
        _add5_body,
        grid=(N // TN,),
        in_specs=[spec] * 5,
        out_specs=spec,
        out_shape=jax.ShapeDtypeStruct((N, 128), jnp.float32),
    )(a, *ps)


def _readout_body(a_ref, p0_ref, p1_ref, p2_ref, p3_ref, ngi_ref, o1w_ref,
                  o1b_ref, o2w_ref, o2b_ref, lw_ref, lb_ref, out_ref, acc_ref):
    j = pl.program_id(0)

    @pl.when(j == 0)
    def _():
        acc_ref[...] = jnp.zeros_like(acc_ref)

    a = a_ref[...] + ((p0_ref[...] + p1_ref[...]) + (p2_ref[...] + p3_ref[...]))
    g = ngi_ref[0]
    oh = (lax.broadcasted_iota(jnp.int32, (G, TN), 0) == g).astype(jnp.float32)
    acc_ref[...] += jnp.dot(oh, a, preferred_element_type=jnp.float32)

    @pl.when(j == pl.num_programs(0) - 1)
    def _():
        m = jax.nn.relu(jnp.dot(acc_ref[...], o1w_ref[...],
                                preferred_element_type=jnp.float32) + o1b_ref[0:1, :])
        m = jax.nn.relu(jnp.dot(m, o2w_ref[...],
                                preferred_element_type=jnp.float32) + o2b_ref[0:1, :])
        out_ref[...] = jnp.dot(m, lw_ref[...],
                               preferred_element_type=jnp.float32) + lb_ref[0:1, :]


def _readout(a, ps, ngi3, o1w, o1b, o2w, o2b, lw_pad, lb_pad):
    spec = pl.BlockSpec((TN, 128), lambda i: (i, 0))
    return pl.pallas_call(
        _readout_body,
        grid=(N // TN,),
        in_specs=[spec, spec, spec, spec, spec,
                  pl.BlockSpec((1, 1, TN), lambda i: (i, 0, 0)),
                  pl.BlockSpec((128, 128), lambda i: (0, 0)),
                  pl.BlockSpec((1, 128), lambda i: (0, 0)),
                  pl.BlockSpec((128, 64), lambda i: (0, 0)),
                  pl.BlockSpec((1, 64), lambda i: (0, 0)),
                  pl.BlockSpec((64, 128), lambda i: (0, 0)),
                  pl.BlockSpec((1, 128), lambda i: (0, 0))],
        out_specs=pl.BlockSpec((G, 128), lambda i: (0, 0)),
        out_shape=jax.ShapeDtypeStruct((G, 128), jnp.float32),
        scratch_shapes=[pltpu.VMEM((G, 128), jnp.float32)],
    )(a, *ps, ngi3, o1w, o1b, o2w, o2b, lw_pad, lb_pad)


def _make_gather(e0):
    def _gather_body(table, src_hbm, dst_hbm, srows_out, drows_out,
                     idx_s, idx_d, rows_s, rows_d,
                     sem_g0, sem_g1, sem_os0, sem_os1, sem_od0, sem_od1):
        wid = lax.axis_index("s") * NC + lax.axis_index("c")
        wloc = wid * EPW
        pltpu.sync_copy(src_hbm.at[pl.ds(e0 + wloc, EPW)], idx_s)
        pltpu.sync_copy(dst_hbm.at[pl.ds(e0 + wloc, EPW)], idx_d)
        sem_g = (sem_g0, sem_g1)
        sem_os = (sem_os0, sem_os1)
        sem_od = (sem_od0, sem_od1)

        def out_start(g, b):
            gb = pl.ds(wloc + g * GED, GED)
            pltpu.async_copy(rows_s.at[b], srows_out.at[gb, :], sem_os[b])
            pltpu.async_copy(rows_d.at[b], drows_out.at[gb, :], sem_od[b])

        def out_wait(g, b):
            gb = pl.ds(wloc + g * GED, GED)
            pltpu.make_async_copy(rows_s.at[b], srows_out.at[gb, :],
                                  sem_os[b]).wait()
            pltpu.make_async_copy(rows_d.at[b], drows_out.at[gb, :],
                                  sem_od[b]).wait()

        def g_start(g, b):
            for j in range(CPG):
                sl = pl.ds(g * GED + j * C, C)
                bl = pl.ds(j * C, C)
                pltpu.async_copy(table.at[idx_s.at[sl]], rows_s.at[b, bl, :],
                                 sem_g[b])
                pltpu.async_copy(table.at[idx_d.at[sl]], rows_d.at[b, bl, :],
                                 sem_g[b])

        def g_wait(g, b):
            for j in range(CPG):
                sl = pl.ds(g * GED + j * C, C)
                bl = pl.ds(j * C, C)
                pltpu.make_async_copy(table.at[idx_s.at[sl]],
                                      rows_s.at[b, bl, :], sem_g[b]).wait()
                pltpu.make_async_copy(table.at[idx_d.at[sl]],
                                      rows_d.at[b, bl, :], sem_g[b]).wait()

        g_start(0, 0)

        def pair(gg, _):
            for b in range(2):
                g = gg * 2 + b

                @pl.when(g < NG)
                def _():
                    @pl.when(g + 1 < NG)
                    def _():
                        @pl.when(g >= 1)
                        def _():
                            out_wait(g - 1, 1 - b)

                        g_start(g + 1, 1 - b)

                    g_wait(g, b)
                    out_start(g, b)
            return ()

        lax.fori_loop(0, (NG + 1) // 2, pair, (), unroll=False)
        out_wait(NG - 2, (NG - 2) % 2)
        out_wait(NG - 1, (NG - 1) % 2)

    return pl.kernel(
        _gather_body,
        out_type=(jax.ShapeDtypeStruct((EC, 128), jnp.float32),
                  jax.ShapeDtypeStruct((EC, 128), jnp.float32)),
        mesh=_sc_mesh(),
        scratch_types=[
            pltpu.VMEM((EPW,), jnp.int32),
            pltpu.VMEM((EPW,), jnp.int32),
            pltpu.VMEM((2, GED, 128), jnp.float32),
            pltpu.VMEM((2, GED, 128), jnp.float32),
            pltpu.SemaphoreType.DMA,
            pltpu.SemaphoreType.DMA,
            pltpu.SemaphoreType.DMA,
            pltpu.SemaphoreType.DMA,
            pltpu.SemaphoreType.DMA,
            pltpu.SemaphoreType.DMA,
        ],
    )


@functools.lru_cache(maxsize=None)
def _gather_chunk(e0):
    return _make_gather(e0)


def _gather_call(e0, table, src, dst):
    return _gather_chunk(e0)(table, src, dst)


NBUF = 2


def _make_scatter():
    def _scatter_body(msg_hbm, dst_hbm, zeros_hbm, p0_out, p1_out,
                      acc, rows, idx0, idx1, sem_m0, sem_m1, sem_i0, sem_i1):
        cid = lax.axis_index("c")
        sid = lax.axis_index("s")
        wid = sid * NC + cid

        @pl.when(sid == 0)
        def _():
            pltpu.sync_copy(zeros_hbm, acc)

        plsc.subcore_barrier()

        sems_m = (sem_m0, sem_m1)
        sems_i = (sem_i0, sem_i1)
        idxs = (idx0, idx1)
        s0 = wid * SPW

        def load(s, b):
            pltpu.async_copy(msg_hbm.at[pl.ds(s * SW, SW), :],
                             rows.at[b], sems_m[b])
            pltpu.async_copy(dst_hbm.at[pl.ds(s * SW, SW)], idxs[b], sems_i[b])

        def wait(s, b):
            pltpu.make_async_copy(msg_hbm.at[pl.ds(s * SW, SW), :],
                                  rows.at[b], sems_m[b]).wait()
            pltpu.make_async_copy(dst_hbm.at[pl.ds(s * SW, SW)], idxs[b],
                                  sems_i[b]).wait()

        def add(b):
            pltpu.sync_copy(rows.at[b], acc.at[idxs[b]], add=True)

        for b in range(NBUF):
            load(s0 + b, b)

        def group(g, _):
            for b in range(NBUF):
                it = g * NBUF + b
                s = s0 + it
                wait(s, b)
                add(b)

                @pl.when(it + NBUF < SPW)
                def _():
                    load(s + NBUF, b)
            return ()

        lax.fori_loop(0, SPW // NBUF, group, (), unroll=False)

        s = s0 + SPW - 1
        wait(s, (SPW - 1) % NBUF)
        add((SPW - 1) % NBUF)

        @pl.when(wid < SLEFT)
        def _():
            s = NW * SPW + wid
            load(s, 0)
            wait(s, 0)
            add(0)

        plsc.subcore_barrier()

        slab = 624
        sl = pl.ds(sid * slab, slab)

        @pl.when(cid == 0)
        def _():
            pltpu.sync_copy(acc.at[sl, :], p0_out.at[sl, :])

            @pl.when(sid == NS - 1)
            def _():
                tl = pl.ds(NS * slab, N - NS * slab)
                pltpu.sync_copy(acc.at[tl, :], p0_out.at[tl, :])

        @pl.when(cid == 1)
        def _():
            pltpu.sync_copy(acc.at[sl, :], p1_out.at[sl, :])

            @pl.when(sid == NS - 1)
            def _():
                tl = pl.ds(NS * slab, N - NS * slab)
                pltpu.sync_copy(acc.at[tl, :], p1_out.at[tl, :])

    return pl.kernel(
        _scatter_body,
        out_type=(jax.ShapeDtypeStruct((N, 128), jnp.float32),
                  jax.ShapeDtypeStruct((N, 128), jnp.float32)),
        mesh=_sc_mesh(),
        scratch_types=[
            pltpu.VMEM_SHARED((N, 128), jnp.float32),
            pltpu.VMEM((NBUF, SW, 128), jnp.float32),
            pltpu.VMEM((SW,), jnp.int32),
            pltpu.VMEM((SW,), jnp.int32),
            pltpu.SemaphoreType.DMA,
            pltpu.SemaphoreType.DMA,
            pltpu.SemaphoreType.DMA,
            pltpu.SemaphoreType.DMA,
        ],
    )


@functools.lru_cache(maxsize=None)
def _scatter_kernel():
    return _make_scatter()


def _scatter_call(msg_c, dst_c, zeros_n):
    return _scatter_kernel()(msg_c, dst_c, zeros_n)


def kernel(atom_types, bond_types, node_graph_indices, connectivity,
           atom_emb, bond_emb,
           abn_g, abn_b, abn_m, abn_v,
           bbn_g, bbn_b, bbn_m, bbn_v,
           bu1_W, bu2_W, bu2_b, au_W,
           o1_W, o1_b, o2_W, o2_b, last_W, last_b):
    f32 = jnp.float32
    dst = connectivity[:, 0].astype(jnp.int32)
    src = connectivity[:, 1].astype(jnp.int32)
    dst_c = [lax.slice_in_dim(dst, c * EC, (c + 1) * EC) for c in range(NCHUNK)]
    at3 = atom_types.astype(jnp.int32).reshape(N // TN, 1, TN)
    bt3 = bond_types.astype(jnp.int32).reshape(E // TE, 1, TE)
    bt3_c = [bt3[c * (EC // TE):(c + 1) * (EC // TE)] for c in range(NCHUNK)]
    ngi3 = node_graph_indices.astype(jnp.int32).reshape(N // TN, 1, TN)

    aemb_pad = jnp.zeros((128, 128), f32).at[:atom_emb.shape[0]].set(atom_emb)
    bemb_pad = jnp.zeros((128, 128), f32).at[:bond_emb.shape[0]].set(bond_emb)
    lw_pad = jnp.zeros((64, 128), f32).at[:, :1].set(last_W)
    lb_pad = jnp.zeros((128,), f32).at[:1].set(last_b).reshape(1, 128)
    zeros_n = jnp.zeros((N, 128), f32)

    a_sc = abn_g / jnp.sqrt(abn_v + 1e-3)
    a_sh = abn_b - abn_m * a_sc
    b_sc = bbn_g / jnp.sqrt(bbn_v + 1e-3)
    b_sh = bbn_b - bbn_m * b_sc

    atom_state = _embed(at3, aemb_pad, N, TN)

    bond_c = [None] * NCHUNK
    for i in range(L):
        w1 = bu1_W[i]
        w1p = jnp.concatenate([
            w1[0:128] * a_sc[i][:, None],
            w1[128:256] * a_sc[i][:, None],
            w1[256:384] * b_sc[i][:, None],
        ]).astype(jnp.bfloat16)
        zc = (a_sh[i] @ (w1[0:128] + w1[128:256])
              + b_sh[i] @ w1[256:384]).reshape(1, 2 * D)
        aup = (au_W[i] * a_sc[i][:, None]).astype(jnp.bfloat16)
        suc = (a_sh[i] @ au_W[i]).reshape(1, D)
        wargs = (w1p, bu2_W[i], bu2_b[i].reshape(1, D), zc, aup, suc)

        rows_c = [_gather_call(c * EC, atom_state, src, dst)
                  for c in range(NCHUNK)]
        msg_c = [None] * NCHUNK
        for c in range(NCHUNK):
            srows, drows = rows_c[c]
            if i == 0:
                bond_c[c], msg_c[c] = _edge_layer0(
                    srows, drows, bt3_c[c], bemb_pad, wargs)
            else:
                bond_c[c], msg_c[c] = _edge_layer(
                    srows, drows, bond_c[c], wargs)
        parts = []
        for c in range(NCHUNK):
            parts.extend(_scatter_call(msg_c[c], dst_c[c], zeros_n))
        if i < L - 1:
            atom_state = _add5(atom_state, parts)

    out = _readout(atom_state, parts, ngi3,
                   o1_W, o1_b.reshape(1, 128), o2_W, o2_b.reshape(1, 64),
                   lw_pad, lb_pad)
    return out[:, :1]
